# baseline (speedup 1.0000x reference)
import jax
import jax.numpy as jnp
from jax import lax
from jax.experimental import pallas as pl
from jax.experimental.pallas import tpu as pltpu
from jax.experimental.pallas import tpu_sc as plsc

NUM_CORES = 2
NUM_SUBCORES = 16
NW = NUM_CORES * NUM_SUBCORES

CHUNK = 640
D = 32


NSLOT = 4


def _gather_body(idx_hbm, table_hbm, out_hbm, idx_v, *rest):
    bufs = rest[:NSLOT]
    gsems = rest[NSLOT:2 * NSLOT]
    ssems = rest[2 * NSLOT:3 * NSLOT]
    wid = lax.axis_index("s") * NUM_CORES + lax.axis_index("c")
    n_chunks = idx_hbm.shape[1]

    pltpu.sync_copy(idx_hbm.at[wid], idx_v)

    for k in range(NSLOT):
        pltpu.async_copy(table_hbm.at[idx_v.at[k]], bufs[k], gsems[k])

    @pl.loop(0, n_chunks - NSLOT, step=NSLOT)
    def _(base):
        for k in range(NSLOT):
            pltpu.make_async_copy(table_hbm.at[idx_v.at[base + k]], bufs[k], gsems[k]).wait()
            pltpu.async_copy(bufs[k], out_hbm.at[wid, base + k], ssems[k])
        for k in range(NSLOT):
            pltpu.make_async_copy(bufs[k], out_hbm.at[wid, base + k], ssems[k]).wait()
            pltpu.async_copy(
                table_hbm.at[idx_v.at[base + NSLOT + k]], bufs[k], gsems[k])

    base = n_chunks - NSLOT
    for k in range(NSLOT):
        pltpu.make_async_copy(table_hbm.at[idx_v.at[base + k]], bufs[k], gsems[k]).wait()
        pltpu.async_copy(bufs[k], out_hbm.at[wid, base + k], ssems[k])
    for k in range(NSLOT):
        pltpu.make_async_copy(bufs[k], out_hbm.at[wid, base + k], ssems[k]).wait()


def kernel(embed_id, weight):
    batch, hist = embed_id.shape
    total = batch * hist
    assert total % (NW * CHUNK) == 0
    n_chunks = total // (NW * CHUNK)
    assert n_chunks % NSLOT == 0

    idx3 = embed_id.astype(jnp.int32).reshape(NW, n_chunks, CHUNK)

    mesh = plsc.VectorSubcoreMesh(
        core_axis_name="c", subcore_axis_name="s",
        num_cores=NUM_CORES, num_subcores=NUM_SUBCORES,
    )
    run = pl.kernel(
        _gather_body,
        out_type=jax.ShapeDtypeStruct((NW, n_chunks, CHUNK, D), jnp.float32),
        mesh=mesh,
        compiler_params=pltpu.CompilerParams(use_tc_tiling_on_sc=False),
        scratch_types=(
            [pltpu.VMEM((n_chunks, CHUNK), jnp.int32)]
            + [pltpu.VMEM((CHUNK, D), jnp.float32) for _ in range(NSLOT)]
            + [pltpu.SemaphoreType.DMA for _ in range(2 * NSLOT)]
        ),
    )
    out = run(idx3, weight)
    return out.reshape(batch, hist, D)
